# baseline (speedup 1.0000x reference)
import jax
import jax.numpy as jnp
from jax import lax
from jax.experimental import pallas as pl
from jax.experimental.pallas import tpu as pltpu
from jax.experimental.pallas import tpu_sc as plsc

_BATCH = 4096
_FLAT = 200 * 64
_NC = 2
_NS = 16
_NW = _NC * _NS
_ROWS_PER_W = _FLAT // _NW
_CHUNK = 8
_NBUF = 3
_NCHUNK = _ROWS_PER_W // _CHUNK
_NMAIN = (_NCHUNK // _NBUF) * _NBUF
_NVEC = _BATCH // 16


def _sc_add(x_hbm, pos_hbm, out_hbm, pos_v, b0, b1, b2,
            si0, si1, si2, so0, so1, so2):
    bufs = (b0, b1, b2)
    sin = (si0, si1, si2)
    sout = (so0, so1, so2)
    wid = lax.axis_index("s") * _NC + lax.axis_index("c")
    base = wid * _ROWS_PER_W
    pltpu.sync_copy(pos_hbm.at[pl.ds(base * 16, _ROWS_PER_W * 16)], pos_v)

    def rows_of(ci):
        return pl.ds(base + ci * _CHUNK, _CHUNK)

    for b in range(_NBUF - 1):
        pltpu.async_copy(x_hbm.at[rows_of(b)], bufs[b], sin[b])

    def step(ci, b):
        buf = bufs[b]
        bp = (b - 1) % _NBUF
        pltpu.make_async_copy(x_hbm.at[rows_of(ci)], buf, sin[b]).wait()

        for k in range(_CHUNK):
            pk = pos_v[pl.ds((ci * _CHUNK + k) * 16, 16)]

            def inner(v, c):
                sl = pl.ds(v * 16, 16)
                buf[k, sl] = buf[k, sl] + pk
                return c

            lax.fori_loop(0, _NVEC, inner, 0, unroll=8)

        pltpu.async_copy(buf, out_hbm.at[rows_of(ci)], sout[b])

        @pl.when(ci >= 1)
        def _():
            pltpu.make_async_copy(
                bufs[bp], out_hbm.at[rows_of(ci - 1)], sout[bp]).wait()

        @pl.when(ci + _NBUF - 1 < _NCHUNK)
        def _():
            pltpu.async_copy(
                x_hbm.at[rows_of(ci + _NBUF - 1)], bufs[bp], sin[bp])

    def outer(g, carry):
        for b in range(_NBUF):
            step(g * _NBUF + b, b)
        return carry

    lax.fori_loop(0, _NMAIN // _NBUF, outer, 0)
    for ci in range(_NMAIN, _NCHUNK):
        step(ci, ci % _NBUF)
    pltpu.make_async_copy(
        bufs[(_NCHUNK - 1) % _NBUF],
        out_hbm.at[rows_of(_NCHUNK - 1)],
        sout[(_NCHUNK - 1) % _NBUF]).wait()


def kernel(x, pos_table):
    batch, maxlen, embed = x.shape
    xt = x.transpose(1, 2, 0).reshape(_FLAT, batch)
    pos_rep = jnp.broadcast_to(
        pos_table.reshape(_FLAT, 1), (_FLAT, 16)).reshape(_FLAT * 16)
    mesh = plsc.VectorSubcoreMesh(core_axis_name="c", subcore_axis_name="s")
    f = pl.kernel(
        _sc_add,
        mesh=mesh,
        out_type=jax.ShapeDtypeStruct((_FLAT, batch), jnp.float32),
        scratch_types=[
            pltpu.VMEM((_ROWS_PER_W * 16,), jnp.float32),
            pltpu.VMEM((_CHUNK, _BATCH), jnp.float32),
            pltpu.VMEM((_CHUNK, _BATCH), jnp.float32),
            pltpu.VMEM((_CHUNK, _BATCH), jnp.float32),
            pltpu.SemaphoreType.DMA,
            pltpu.SemaphoreType.DMA,
            pltpu.SemaphoreType.DMA,
            pltpu.SemaphoreType.DMA,
            pltpu.SemaphoreType.DMA,
            pltpu.SemaphoreType.DMA,
        ],
    )
    out_t = f(xt, pos_rep)
    return out_t.reshape(maxlen, embed, batch).transpose(2, 0, 1)


# speedup vs baseline: 1.3871x; 1.3871x over previous
import jax
import jax.numpy as jnp
from jax.experimental import pallas as pl

_F_BLK = 512
_POS_ROWS = _F_BLK // 128


def _add_body(x_ref, pos_ref, o_ref):
    o_ref[...] = x_ref[...] + jnp.transpose(pos_ref[0], (1, 0))


def kernel(x, pos_table):
    batch, maxlen, embed = x.shape
    flat = maxlen * embed
    xt = x.transpose(1, 2, 0).reshape(flat, batch)
    post = pos_table.reshape(flat // _F_BLK, 1, _F_BLK)

    grid = (flat // _F_BLK,)
    out_t = pl.pallas_call(
        _add_body,
        grid=grid,
        in_specs=[
            pl.BlockSpec((_F_BLK, batch), lambda i: (i, 0)),
            pl.BlockSpec((1, 1, _F_BLK), lambda i: (i, 0, 0)),
        ],
        out_specs=pl.BlockSpec((_F_BLK, batch), lambda i: (i, 0)),
        out_shape=jax.ShapeDtypeStruct((flat, batch), x.dtype),
    )(xt, post)
    return out_t.reshape(maxlen, embed, batch).transpose(2, 0, 1)
